# baseline (speedup 1.0000x reference)
import functools

import jax
import jax.numpy as jnp
from jax import lax
from jax.experimental import pallas as pl
from jax.experimental.pallas import tpu as pltpu
from jax.experimental.pallas import tpu_sc as plsc

N_NODES = 10000
N_EDGES = 320000
D_HID = 128

NC = 2
NS = 16
NW = NC * NS
EK = 128
NCH = 80
EPW = NCH * EK
E_PAD = NW * EPW
N_PAD = 10240
ROWS_PER_SUB = N_PAD // NS

_sc_mesh = functools.partial(
    plsc.VectorSubcoreMesh, core_axis_name="c", subcore_axis_name="s")


@functools.partial(
    pl.kernel,
    mesh=_sc_mesh(),
    out_type=jax.ShapeDtypeStruct((NC * N_NODES,), jnp.float32),
    scratch_types=[
        pltpu.VMEM((NCH, EK), jnp.int32),
        pltpu.VMEM((NCH, EK), jnp.float32),
        pltpu.VMEM((N_NODES,), jnp.float32),
        pltpu.VMEM_SHARED((N_NODES,), jnp.float32),
        pltpu.SemaphoreType.DMA,
    ],
)
def _deg_kernel(dst_hbm, ew_hbm, zeros_hbm, out_hbm, dst_v, w_v, deg_v,
                acc_sh, sem):
    c = lax.axis_index("c")
    s = lax.axis_index("s")
    wid = s * NC + c

    crows = pl.ds(wid * NCH, NCH)
    pltpu.sync_copy(dst_hbm.at[crows], dst_v)
    pltpu.sync_copy(ew_hbm.at[crows], w_v)

    @pl.when(s == 0)
    def _():
        pltpu.sync_copy(zeros_hbm, deg_v)
        pltpu.sync_copy(deg_v, acc_sh)

    plsc.subcore_barrier()

    def fire(i, carry):
        pltpu.async_copy(w_v.at[i], acc_sh.at[dst_v.at[i]], sem, add=True)
        return carry

    lax.fori_loop(0, NCH, fire, 0)

    def drain(i, carry):
        pltpu.make_async_copy(w_v.at[i], acc_sh.at[dst_v.at[i]], sem).wait()
        return carry

    lax.fori_loop(0, NCH, drain, 0)
    plsc.subcore_barrier()

    @pl.when(s == 0)
    def _():
        pltpu.sync_copy(acc_sh, deg_v)
        pltpu.sync_copy(deg_v, out_hbm.at[pl.ds(c * N_NODES, N_NODES)])


def _scale_rows(rows_v, w_v, ci):
    def group(g, carry):
        wv = w_v[ci, pl.ds(g * 16, 16)]
        for k in range(16):
            e = g * 16 + k
            w = wv[k]
            for j in range(D_HID // 16):
                sl = pl.ds(j * 16, 16)
                rows_v[e, sl] = rows_v[e, sl] * w
        return carry

    lax.fori_loop(0, EK // 16, group, 0)


@functools.partial(
    pl.kernel,
    mesh=_sc_mesh(),
    out_type=jax.ShapeDtypeStruct((NC, N_PAD, D_HID), jnp.float32),
    scratch_types=[
        pltpu.VMEM((NCH, EK), jnp.int32),
        pltpu.VMEM((NCH, EK), jnp.int32),
        pltpu.VMEM((NCH, EK), jnp.float32),
        pltpu.VMEM((EK, D_HID), jnp.float32),
        pltpu.VMEM((EK, D_HID), jnp.float32),
        pltpu.VMEM_SHARED((N_PAD, D_HID), jnp.float32),
        pltpu.SemaphoreType.DMA,
        pltpu.SemaphoreType.DMA,
        pltpu.SemaphoreType.DMA,
        pltpu.SemaphoreType.DMA,
    ],
)
def _agg_kernel(g_hbm, src_hbm, dst_hbm, ew_hbm, zeros_hbm, out_hbm,
                src_v, dst_v, w_v, rows_a, rows_b, acc_sh,
                gsa, gsb, ssa, ssb):
    c = lax.axis_index("c")
    s = lax.axis_index("s")
    wid = s * NC + c

    crows = pl.ds(wid * NCH, NCH)
    pltpu.sync_copy(src_hbm.at[crows], src_v)
    pltpu.sync_copy(dst_hbm.at[crows], dst_v)
    pltpu.sync_copy(ew_hbm.at[crows], w_v)

    n_slab = ROWS_PER_SUB // EK
    pltpu.sync_copy(zeros_hbm, rows_a)
    for t in range(n_slab):
        pltpu.sync_copy(
            rows_a, acc_sh.at[pl.ds(s * ROWS_PER_SUB + t * EK, EK)])
    plsc.subcore_barrier()

    def gather(ci, buf, gsem):
        return pltpu.async_copy(g_hbm.at[src_v.at[ci]], buf, gsem)

    def scatter(ci, buf, ssem):
        return pltpu.async_copy(buf, acc_sh.at[dst_v.at[ci]], ssem, add=True)

    def gather_wait(ci, buf, gsem):
        pltpu.make_async_copy(g_hbm.at[src_v.at[ci]], buf, gsem).wait()

    def scatter_wait(ci, buf, ssem):
        pltpu.make_async_copy(buf, acc_sh.at[dst_v.at[ci]], ssem).wait()

    def body(i, carry):
        pltpu.async_copy(g_hbm.at[src_v.at[i]], rows_a, gsa).wait()
        _scale_rows(rows_a, w_v, i)
        pltpu.sync_copy(rows_a, acc_sh.at[dst_v.at[i]], add=True)
        return carry

    lax.fori_loop(0, NCH, body, 0)
    plsc.subcore_barrier()
    for t in range(n_slab):
        rs = pl.ds(s * ROWS_PER_SUB + t * EK, EK)
        pltpu.sync_copy(acc_sh.at[rs], rows_a)
        pltpu.sync_copy(rows_a, out_hbm.at[c, rs])


_ROWS = 400
_GRID = N_NODES // _ROWS


def _k0_body(x_ref, w_ref, degp_ref, dinv_ref, g_ref):
    deg = degp_ref[0] + degp_ref[1] + 1.0
    dinv = jnp.where(deg > 0, lax.rsqrt(deg), 0.0)
    dinv_ref[...] = dinv
    g_ref[...] = dinv * jnp.dot(x_ref[...], w_ref[...],
                                preferred_element_type=jnp.float32)


def _kmid_body(p_ref, gp_ref, dinv_ref, b_ref, w_ref, g_ref, *, relu):
    dinv = dinv_ref[...]
    h = dinv * (p_ref[0] + p_ref[1] + gp_ref[...]) + b_ref[...]
    if relu:
        h = jnp.maximum(h, 0.0)
    g_ref[...] = dinv * jnp.dot(h, w_ref[...],
                                preferred_element_type=jnp.float32)


def _kelem_body(p_ref, gp_ref, dinv_ref, b_ref, u_ref):
    dinv = dinv_ref[...]
    u_ref[...] = dinv * (dinv * (p_ref[0] + p_ref[1] + gp_ref[...])
                         + b_ref[...])


def _kfin_body(p_ref, u_ref, dinv_ref, w_ref, b_ref, o_ref):
    o_ref[...] = dinv_ref[...] * jnp.dot(
        p_ref[0] + p_ref[1] + u_ref[...], w_ref[...],
        preferred_element_type=jnp.float32) + b_ref[...]


def _row_spec(d):
    return pl.BlockSpec((_ROWS, d), lambda i: (i, 0))


def _part_spec(d):
    return pl.BlockSpec((NC, _ROWS, d), lambda i: (0, i, 0))


def _full_spec(r, c):
    return pl.BlockSpec((r, c), lambda i: (0, 0))


def _k0(x, w0, degp):
    return pl.pallas_call(
        _k0_body,
        grid=(_GRID,),
        in_specs=[_row_spec(D_HID), _full_spec(D_HID, D_HID), _part_spec(1)],
        out_specs=[_row_spec(1), _row_spec(D_HID)],
        out_shape=[jax.ShapeDtypeStruct((N_NODES, 1), jnp.float32),
                   jax.ShapeDtypeStruct((N_NODES, D_HID), jnp.float32)],
    )(x, w0, degp)


def _kmid(p, gp, dinv, b, w, relu):
    return pl.pallas_call(
        functools.partial(_kmid_body, relu=relu),
        grid=(_GRID,),
        in_specs=[_part_spec(D_HID), _row_spec(D_HID), _row_spec(1),
                  _full_spec(1, D_HID), _full_spec(D_HID, w.shape[1])],
        out_specs=_row_spec(w.shape[1]),
        out_shape=jax.ShapeDtypeStruct((N_NODES, w.shape[1]), jnp.float32),
    )(p, gp, dinv, b, w)


def _kelem(p, gp, dinv, b):
    return pl.pallas_call(
        _kelem_body,
        grid=(_GRID,),
        in_specs=[_part_spec(D_HID), _row_spec(D_HID), _row_spec(1),
                  _full_spec(1, D_HID)],
        out_specs=_row_spec(D_HID),
        out_shape=jax.ShapeDtypeStruct((N_NODES, D_HID), jnp.float32),
    )(p, gp, dinv, b)


def _kfin(p, u, dinv, w2, b2):
    ncls = w2.shape[1]
    return pl.pallas_call(
        _kfin_body,
        grid=(_GRID,),
        in_specs=[_part_spec(D_HID), _row_spec(D_HID), _row_spec(1),
                  _full_spec(D_HID, ncls), _full_spec(1, ncls)],
        out_specs=_row_spec(ncls),
        out_shape=jax.ShapeDtypeStruct((N_NODES, ncls), jnp.float32),
    )(p, u, dinv, w2, b2)


def kernel(x, edge_index, edge_attr, W0, b0, W1, b1, W2, b2):
    pad = E_PAD - N_EDGES
    src = jnp.concatenate(
        [edge_index[0], jnp.zeros((pad,), jnp.int32)]).reshape(NW * NCH, EK)
    dst = jnp.concatenate(
        [edge_index[1], jnp.zeros((pad,), jnp.int32)]).reshape(NW * NCH, EK)
    ew = jnp.concatenate(
        [edge_attr, jnp.zeros((pad,), jnp.float32)]).reshape(NW * NCH, EK)

    zeros_n = jnp.zeros((N_NODES,), jnp.float32)
    zeros128 = jnp.zeros((EK, D_HID), jnp.float32)

    degp = _deg_kernel(dst, ew, zeros_n).reshape(NC, N_NODES, 1)

    dinv, g0 = _k0(x, W0, degp)
    p0 = _agg_kernel(g0, src, dst, ew, zeros128)
    g1 = _kmid(p0, g0, dinv, b0.reshape(1, D_HID), W1, relu=True)
    p1 = _agg_kernel(g1, src, dst, ew, zeros128)
    u2 = _kelem(p1, g1, dinv, b1.reshape(1, D_HID))
    p2 = _agg_kernel(u2, src, dst, ew, zeros128)
    return _kfin(p2, u2, dinv, W2, b2.reshape(1, W2.shape[1]))


# speedup vs baseline: 1.3018x; 1.3018x over previous
import functools

import jax
import jax.numpy as jnp
from jax import lax
from jax.experimental import pallas as pl
from jax.experimental.pallas import tpu as pltpu
from jax.experimental.pallas import tpu_sc as plsc

N_NODES = 10000
N_EDGES = 320000
D_HID = 128

NC = 2
NS = 16
NW = NC * NS
EPW = N_EDGES // NW
EK = 80
N_PAD = 10240
ROWS_PER_SUB = N_PAD // NS

_sc_mesh = functools.partial(
    plsc.VectorSubcoreMesh, core_axis_name="c", subcore_axis_name="s")


@functools.partial(
    pl.kernel,
    mesh=_sc_mesh(),
    out_type=jax.ShapeDtypeStruct((NC * N_NODES,), jnp.float32),
    scratch_types=[
        pltpu.VMEM((EK,), jnp.int32),
        pltpu.VMEM((EK,), jnp.float32),
        pltpu.VMEM((N_NODES,), jnp.float32),
        pltpu.VMEM_SHARED((N_NODES,), jnp.float32),
    ],
)
def _deg_kernel(dst_hbm, ew_hbm, zeros_hbm, out_hbm, dst_v, w_v, deg_v,
                acc_sh):
    c = lax.axis_index("c")
    s = lax.axis_index("s")
    wid = s * NC + c

    @pl.when(s == 0)
    def _():
        pltpu.sync_copy(zeros_hbm, deg_v)
        pltpu.sync_copy(deg_v, acc_sh)

    plsc.subcore_barrier()

    def chunk(i, carry):
        base = wid * EPW + i * EK
        pltpu.sync_copy(dst_hbm.at[pl.ds(base, EK)], dst_v)
        pltpu.sync_copy(ew_hbm.at[pl.ds(base, EK)], w_v)
        pltpu.sync_copy(w_v, acc_sh.at[dst_v], add=True)
        return carry

    lax.fori_loop(0, EPW // EK, chunk, 0)
    plsc.subcore_barrier()

    @pl.when(s == 0)
    def _():
        pltpu.sync_copy(acc_sh, deg_v)
        pltpu.sync_copy(deg_v, out_hbm.at[pl.ds(c * N_NODES, N_NODES)])


@functools.partial(
    pl.kernel,
    mesh=_sc_mesh(),
    out_type=jax.ShapeDtypeStruct((NC, N_PAD, D_HID), jnp.float32),
    scratch_types=[
        pltpu.VMEM((8, EK), jnp.int32),
        pltpu.VMEM((EK,), jnp.float32),
        pltpu.VMEM((EK, D_HID), jnp.float32),
        pltpu.VMEM_SHARED((N_PAD, D_HID), jnp.float32),
        pltpu.SemaphoreType.DMA,
    ],
)
def _agg_kernel(g_hbm, ed_hbm, ew_hbm, zeros_hbm, out_hbm,
                ed_v, w_v, rows_v, acc_sh, sem):
    c = lax.axis_index("c")
    s = lax.axis_index("s")
    wid = s * NC + c
    n_slab = ROWS_PER_SUB // EK
    pltpu.sync_copy(zeros_hbm, rows_v)
    for t in range(n_slab):
        pltpu.sync_copy(
            rows_v, acc_sh.at[pl.ds(s * ROWS_PER_SUB + t * EK, EK)])
    plsc.subcore_barrier()

    def chunk(i, carry):
        crow = (wid * (EPW // EK) + i) * 8
        pltpu.sync_copy(ed_hbm.at[pl.ds(crow, 8)], ed_v)
        pltpu.sync_copy(ew_hbm.at[pl.ds(wid * EPW + i * EK, EK)], w_v)
        pltpu.async_copy(g_hbm.at[ed_v.at[0]], rows_v, sem).wait()

        def scale(g, cc):
            wv = w_v[pl.ds(g * 16, 16)]
            for k in range(16):
                e = g * 16 + k
                w = wv[k]
                for j in range(D_HID // 16):
                    sl = pl.ds(j * 16, 16)
                    rows_v[e, sl] = rows_v[e, sl] * w
            return cc

        lax.fori_loop(0, EK // 16, scale, 0)
        pltpu.sync_copy(rows_v, acc_sh.at[ed_v.at[1]], add=True)
        return carry

    lax.fori_loop(0, EPW // EK, chunk, 0)
    plsc.subcore_barrier()
    for t in range(n_slab):
        rs = pl.ds(s * ROWS_PER_SUB + t * EK, EK)
        pltpu.sync_copy(acc_sh.at[rs], rows_v)
        pltpu.sync_copy(rows_v, out_hbm.at[c, rs])


_ROWS = 400
_GRID = N_NODES // _ROWS


def _k0_body(x_ref, w_ref, degp_ref, dinv_ref, g_ref):
    deg = degp_ref[0] + degp_ref[1] + 1.0
    dinv = jnp.where(deg > 0, lax.rsqrt(deg), 0.0)
    dinv_ref[...] = dinv
    g_ref[...] = dinv * jnp.dot(x_ref[...], w_ref[...],
                                preferred_element_type=jnp.float32)


def _kmid_body(p_ref, gp_ref, dinv_ref, b_ref, w_ref, g_ref, *, relu):
    dinv = dinv_ref[...]
    h = dinv * (p_ref[0] + p_ref[1] + gp_ref[...]) + b_ref[...]
    if relu:
        h = jnp.maximum(h, 0.0)
    g_ref[...] = dinv * jnp.dot(h, w_ref[...],
                                preferred_element_type=jnp.float32)


def _kelem_body(p_ref, gp_ref, dinv_ref, b_ref, u_ref):
    dinv = dinv_ref[...]
    u_ref[...] = dinv * (dinv * (p_ref[0] + p_ref[1] + gp_ref[...])
                         + b_ref[...])


def _kfin_body(p_ref, u_ref, dinv_ref, w_ref, b_ref, o_ref):
    o_ref[...] = dinv_ref[...] * jnp.dot(
        p_ref[0] + p_ref[1] + u_ref[...], w_ref[...],
        preferred_element_type=jnp.float32) + b_ref[...]


def _row_spec(d):
    return pl.BlockSpec((_ROWS, d), lambda i: (i, 0))


def _part_spec(d):
    return pl.BlockSpec((NC, _ROWS, d), lambda i: (0, i, 0))


def _full_spec(r, c):
    return pl.BlockSpec((r, c), lambda i: (0, 0))


def _k0(x, w0, degp):
    return pl.pallas_call(
        _k0_body,
        grid=(_GRID,),
        in_specs=[_row_spec(D_HID), _full_spec(D_HID, D_HID), _part_spec(1)],
        out_specs=[_row_spec(1), _row_spec(D_HID)],
        out_shape=[jax.ShapeDtypeStruct((N_NODES, 1), jnp.float32),
                   jax.ShapeDtypeStruct((N_NODES, D_HID), jnp.float32)],
    )(x, w0, degp)


def _kmid(p, gp, dinv, b, w, relu):
    return pl.pallas_call(
        functools.partial(_kmid_body, relu=relu),
        grid=(_GRID,),
        in_specs=[_part_spec(D_HID), _row_spec(D_HID), _row_spec(1),
                  _full_spec(1, D_HID), _full_spec(D_HID, w.shape[1])],
        out_specs=_row_spec(w.shape[1]),
        out_shape=jax.ShapeDtypeStruct((N_NODES, w.shape[1]), jnp.float32),
    )(p, gp, dinv, b, w)


def _kelem(p, gp, dinv, b):
    return pl.pallas_call(
        _kelem_body,
        grid=(_GRID,),
        in_specs=[_part_spec(D_HID), _row_spec(D_HID), _row_spec(1),
                  _full_spec(1, D_HID)],
        out_specs=_row_spec(D_HID),
        out_shape=jax.ShapeDtypeStruct((N_NODES, D_HID), jnp.float32),
    )(p, gp, dinv, b)


def _kfin(p, u, dinv, w2, b2):
    ncls = w2.shape[1]
    return pl.pallas_call(
        _kfin_body,
        grid=(_GRID,),
        in_specs=[_part_spec(D_HID), _row_spec(D_HID), _row_spec(1),
                  _full_spec(D_HID, ncls), _full_spec(1, ncls)],
        out_specs=_row_spec(ncls),
        out_shape=jax.ShapeDtypeStruct((N_NODES, ncls), jnp.float32),
    )(p, u, dinv, w2, b2)


def kernel(x, edge_index, edge_attr, W0, b0, W1, b1, W2, b2):
    src = edge_index[0]
    dst = edge_index[1]
    ew = edge_attr

    n_chunks = N_EDGES // EK
    ed = jnp.stack(
        [src.reshape(n_chunks, EK), dst.reshape(n_chunks, EK)]
        + [jnp.zeros((n_chunks, EK), jnp.int32)] * 6,
        axis=1).reshape(n_chunks * 8, EK)

    zeros_n = jnp.zeros((N_NODES,), jnp.float32)
    zeros128 = jnp.zeros((EK, D_HID), jnp.float32)

    degp = _deg_kernel(dst, ew, zeros_n).reshape(NC, N_NODES, 1)

    dinv, g0 = _k0(x, W0, degp)
    p0 = _agg_kernel(g0, ed, ew, zeros128)
    g1 = _kmid(p0, g0, dinv, b0.reshape(1, D_HID), W1, relu=True)
    p1 = _agg_kernel(g1, ed, ew, zeros128)
    u2 = _kelem(p1, g1, dinv, b1.reshape(1, D_HID))
    p2 = _agg_kernel(u2, ed, ew, zeros128)
    return _kfin(p2, u2, dinv, W2, b2.reshape(1, W2.shape[1]))


# speedup vs baseline: 1.4691x; 1.1286x over previous
import functools

import jax
import jax.numpy as jnp
from jax import lax
from jax.experimental import pallas as pl
from jax.experimental.pallas import tpu as pltpu
from jax.experimental.pallas import tpu_sc as plsc

N_NODES = 10000
N_EDGES = 320000
D_HID = 128

NC = 2
NS = 16
NW = NC * NS
EPW = N_EDGES // NW
EK = 80
N_PAD = 10240
ROWS_PER_SUB = N_PAD // NS

_sc_mesh = functools.partial(
    plsc.VectorSubcoreMesh, core_axis_name="c", subcore_axis_name="s")


@functools.partial(
    pl.kernel,
    mesh=_sc_mesh(),
    out_type=jax.ShapeDtypeStruct((NC * N_NODES,), jnp.float32),
    scratch_types=[
        pltpu.VMEM((EK,), jnp.int32),
        pltpu.VMEM((EK,), jnp.float32),
        pltpu.VMEM((N_NODES,), jnp.float32),
        pltpu.VMEM_SHARED((N_NODES,), jnp.float32),
    ],
)
def _deg_kernel(dst_hbm, ew_hbm, zeros_hbm, out_hbm, dst_v, w_v, deg_v,
                acc_sh):
    c = lax.axis_index("c")
    s = lax.axis_index("s")
    wid = s * NC + c

    @pl.when(s == 0)
    def _():
        pltpu.sync_copy(zeros_hbm, deg_v)
        pltpu.sync_copy(deg_v, acc_sh)

    plsc.subcore_barrier()

    def chunk(i, carry):
        base = wid * EPW + i * EK
        pltpu.sync_copy(dst_hbm.at[pl.ds(base, EK)], dst_v)
        pltpu.sync_copy(ew_hbm.at[pl.ds(base, EK)], w_v)
        pltpu.sync_copy(w_v, acc_sh.at[dst_v], add=True)
        return carry

    lax.fori_loop(0, EPW // EK, chunk, 0)
    plsc.subcore_barrier()

    @pl.when(s == 0)
    def _():
        pltpu.sync_copy(acc_sh, deg_v)
        pltpu.sync_copy(deg_v, out_hbm.at[pl.ds(c * N_NODES, N_NODES)])


@functools.partial(
    pl.kernel,
    mesh=_sc_mesh(),
    out_type=jax.ShapeDtypeStruct((NC, N_PAD, D_HID), jnp.float32),
    scratch_types=[
        pltpu.VMEM((8, EK), jnp.int32),
        pltpu.VMEM((EK, D_HID), jnp.float32),
        pltpu.VMEM_SHARED((N_PAD, D_HID), jnp.float32),
        pltpu.SemaphoreType.DMA,
    ],
)
def _agg_kernel(g_hbm, ed_hbm, zeros_hbm, out_hbm,
                ed_v, rows_v, acc_sh, sem):
    c = lax.axis_index("c")
    s = lax.axis_index("s")
    wid = s * NC + c
    n_slab = ROWS_PER_SUB // EK
    pltpu.sync_copy(zeros_hbm, rows_v)
    for t in range(n_slab):
        pltpu.sync_copy(
            rows_v, acc_sh.at[pl.ds(s * ROWS_PER_SUB + t * EK, EK)])
    plsc.subcore_barrier()

    def chunk(i, carry):
        crow = (wid * (EPW // EK) + i) * 8
        pltpu.sync_copy(ed_hbm.at[pl.ds(crow, 8)], ed_v)
        pltpu.async_copy(g_hbm.at[ed_v.at[0]], rows_v, sem).wait()

        def scale(g, cc):
            wv = lax.bitcast_convert_type(
                ed_v[2, pl.ds(g * 16, 16)], jnp.float32)
            for k in range(16):
                e = g * 16 + k
                w = wv[k]
                for j in range(D_HID // 16):
                    sl = pl.ds(j * 16, 16)
                    rows_v[e, sl] = rows_v[e, sl] * w
            return cc

        lax.fori_loop(0, EK // 16, scale, 0)
        pltpu.sync_copy(rows_v, acc_sh.at[ed_v.at[1]], add=True)
        return carry

    lax.fori_loop(0, EPW // EK, chunk, 0)
    plsc.subcore_barrier()
    for t in range(n_slab):
        rs = pl.ds(s * ROWS_PER_SUB + t * EK, EK)
        pltpu.sync_copy(acc_sh.at[rs], rows_v)
        pltpu.sync_copy(rows_v, out_hbm.at[c, rs])


_ROWS = 400
_GRID = N_NODES // _ROWS


def _k0_body(x_ref, w_ref, degp_ref, dinv_ref, g_ref):
    deg = degp_ref[0] + degp_ref[1] + 1.0
    dinv = jnp.where(deg > 0, lax.rsqrt(deg), 0.0)
    dinv_ref[...] = dinv
    g_ref[...] = dinv * jnp.dot(x_ref[...], w_ref[...],
                                preferred_element_type=jnp.float32)


def _kmid_body(p_ref, gp_ref, dinv_ref, b_ref, w_ref, g_ref, *, relu):
    dinv = dinv_ref[...]
    h = dinv * (p_ref[0] + p_ref[1] + gp_ref[...]) + b_ref[...]
    if relu:
        h = jnp.maximum(h, 0.0)
    g_ref[...] = dinv * jnp.dot(h, w_ref[...],
                                preferred_element_type=jnp.float32)


def _kelem_body(p_ref, gp_ref, dinv_ref, b_ref, u_ref):
    dinv = dinv_ref[...]
    u_ref[...] = dinv * (dinv * (p_ref[0] + p_ref[1] + gp_ref[...])
                         + b_ref[...])


def _kfin_body(p_ref, u_ref, dinv_ref, w_ref, b_ref, o_ref):
    o_ref[...] = dinv_ref[...] * jnp.dot(
        p_ref[0] + p_ref[1] + u_ref[...], w_ref[...],
        preferred_element_type=jnp.float32) + b_ref[...]


def _row_spec(d):
    return pl.BlockSpec((_ROWS, d), lambda i: (i, 0))


def _part_spec(d):
    return pl.BlockSpec((NC, _ROWS, d), lambda i: (0, i, 0))


def _full_spec(r, c):
    return pl.BlockSpec((r, c), lambda i: (0, 0))


def _k0(x, w0, degp):
    return pl.pallas_call(
        _k0_body,
        grid=(_GRID,),
        in_specs=[_row_spec(D_HID), _full_spec(D_HID, D_HID), _part_spec(1)],
        out_specs=[_row_spec(1), _row_spec(D_HID)],
        out_shape=[jax.ShapeDtypeStruct((N_NODES, 1), jnp.float32),
                   jax.ShapeDtypeStruct((N_NODES, D_HID), jnp.float32)],
    )(x, w0, degp)


def _kmid(p, gp, dinv, b, w, relu):
    return pl.pallas_call(
        functools.partial(_kmid_body, relu=relu),
        grid=(_GRID,),
        in_specs=[_part_spec(D_HID), _row_spec(D_HID), _row_spec(1),
                  _full_spec(1, D_HID), _full_spec(D_HID, w.shape[1])],
        out_specs=_row_spec(w.shape[1]),
        out_shape=jax.ShapeDtypeStruct((N_NODES, w.shape[1]), jnp.float32),
    )(p, gp, dinv, b, w)


def _kelem(p, gp, dinv, b):
    return pl.pallas_call(
        _kelem_body,
        grid=(_GRID,),
        in_specs=[_part_spec(D_HID), _row_spec(D_HID), _row_spec(1),
                  _full_spec(1, D_HID)],
        out_specs=_row_spec(D_HID),
        out_shape=jax.ShapeDtypeStruct((N_NODES, D_HID), jnp.float32),
    )(p, gp, dinv, b)


def _kfin(p, u, dinv, w2, b2):
    ncls = w2.shape[1]
    return pl.pallas_call(
        _kfin_body,
        grid=(_GRID,),
        in_specs=[_part_spec(D_HID), _row_spec(D_HID), _row_spec(1),
                  _full_spec(D_HID, ncls), _full_spec(1, ncls)],
        out_specs=_row_spec(ncls),
        out_shape=jax.ShapeDtypeStruct((N_NODES, ncls), jnp.float32),
    )(p, u, dinv, w2, b2)


def kernel(x, edge_index, edge_attr, W0, b0, W1, b1, W2, b2):
    src = edge_index[0]
    dst = edge_index[1]
    ew = edge_attr

    n_chunks = N_EDGES // EK
    ed = jnp.stack(
        [src.reshape(n_chunks, EK), dst.reshape(n_chunks, EK),
         lax.bitcast_convert_type(ew, jnp.int32).reshape(n_chunks, EK)]
        + [jnp.zeros((n_chunks, EK), jnp.int32)] * 5,
        axis=1).reshape(n_chunks * 8, EK)

    zeros_n = jnp.zeros((N_NODES,), jnp.float32)
    zeros128 = jnp.zeros((EK, D_HID), jnp.float32)

    degp = _deg_kernel(dst, ew, zeros_n).reshape(NC, N_NODES, 1)

    dinv, g0 = _k0(x, W0, degp)
    p0 = _agg_kernel(g0, ed, zeros128)
    g1 = _kmid(p0, g0, dinv, b0.reshape(1, D_HID), W1, relu=True)
    p1 = _agg_kernel(g1, ed, zeros128)
    u2 = _kelem(p1, g1, dinv, b1.reshape(1, D_HID))
    p2 = _agg_kernel(u2, ed, zeros128)
    return _kfin(p2, u2, dinv, W2, b2.reshape(1, W2.shape[1]))


# speedup vs baseline: 1.7557x; 1.1951x over previous
import functools

import jax
import jax.numpy as jnp
from jax import lax
from jax.experimental import pallas as pl
from jax.experimental.pallas import tpu as pltpu
from jax.experimental.pallas import tpu_sc as plsc

N_NODES = 10000
N_EDGES = 320000
D_HID = 128

NC = 2
NS = 16
NW = NC * NS
EPW = N_EDGES // NW
EK = 80
N_PAD = 10240
ROWS_PER_SUB = N_PAD // NS

_sc_mesh = functools.partial(
    plsc.VectorSubcoreMesh, core_axis_name="c", subcore_axis_name="s")


@functools.partial(
    pl.kernel,
    mesh=_sc_mesh(),
    out_type=jax.ShapeDtypeStruct((NC * N_NODES,), jnp.float32),
    scratch_types=[
        pltpu.VMEM((EK,), jnp.int32),
        pltpu.VMEM((EK,), jnp.float32),
        pltpu.VMEM((N_NODES,), jnp.float32),
        pltpu.VMEM_SHARED((N_NODES,), jnp.float32),
    ],
)
def _deg_kernel(dst_hbm, ew_hbm, zeros_hbm, out_hbm, dst_v, w_v, deg_v,
                acc_sh):
    c = lax.axis_index("c")
    s = lax.axis_index("s")
    wid = s * NC + c

    @pl.when(s == 0)
    def _():
        pltpu.sync_copy(zeros_hbm, deg_v)
        pltpu.sync_copy(deg_v, acc_sh)

    plsc.subcore_barrier()

    def chunk(i, carry):
        base = wid * EPW + i * EK
        pltpu.sync_copy(dst_hbm.at[pl.ds(base, EK)], dst_v)
        pltpu.sync_copy(ew_hbm.at[pl.ds(base, EK)], w_v)
        pltpu.sync_copy(w_v, acc_sh.at[dst_v], add=True)
        return carry

    lax.fori_loop(0, EPW // EK, chunk, 0)
    plsc.subcore_barrier()

    @pl.when(s == 0)
    def _():
        pltpu.sync_copy(acc_sh, deg_v)
        pltpu.sync_copy(deg_v, out_hbm.at[pl.ds(c * N_NODES, N_NODES)])


@functools.partial(
    pl.kernel,
    mesh=_sc_mesh(),
    out_type=jax.ShapeDtypeStruct((NC, N_PAD, D_HID), jnp.float32),
    scratch_types=[
        pltpu.VMEM((2, 8, EK), jnp.int32),
        pltpu.VMEM((EK, D_HID), jnp.float32),
        pltpu.VMEM_SHARED((N_PAD, D_HID), jnp.float32),
        pltpu.SemaphoreType.DMA((2,)),
        pltpu.SemaphoreType.DMA,
    ],
)
def _agg_kernel(g_hbm, ed_hbm, zeros_hbm, out_hbm,
                ed2_v, rows_v, acc_sh, isem, sem):
    c = lax.axis_index("c")
    s = lax.axis_index("s")
    wid = s * NC + c
    n_slab = ROWS_PER_SUB // EK
    pltpu.sync_copy(zeros_hbm, rows_v)
    for t in range(n_slab):
        pltpu.sync_copy(
            rows_v, acc_sh.at[pl.ds(s * ROWS_PER_SUB + t * EK, EK)])
    plsc.subcore_barrier()

    def idx_dma(ci, b):
        crow = (wid * (EPW // EK) + ci) * 8
        return pltpu.make_async_copy(
            ed_hbm.at[pl.ds(crow, 8)], ed2_v.at[b], isem.at[b])

    idx_dma(0, 0).start()

    def chunk(i, carry):
        b = lax.rem(i, 2)

        @pl.when(i + 1 < EPW // EK)
        def _():
            idx_dma(i + 1, 1 - b).start()

        idx_dma(i, b).wait()
        ed_v = ed2_v.at[b]
        pltpu.async_copy(g_hbm.at[ed_v.at[0]], rows_v, sem).wait()

        def scale(g, cc):
            wv = lax.bitcast_convert_type(
                ed2_v[b, 2, pl.ds(g * 16, 16)], jnp.float32)
            for k in range(16):
                e = g * 16 + k
                w = wv[k]
                for j in range(D_HID // 16):
                    sl = pl.ds(j * 16, 16)
                    rows_v[e, sl] = rows_v[e, sl] * w
            return cc

        lax.fori_loop(0, EK // 16, scale, 0)
        pltpu.sync_copy(rows_v, acc_sh.at[ed_v.at[1]], add=True)
        return carry

    lax.fori_loop(0, EPW // EK, chunk, 0)
    plsc.subcore_barrier()
    for t in range(n_slab):
        rs = pl.ds(s * ROWS_PER_SUB + t * EK, EK)
        pltpu.sync_copy(acc_sh.at[rs], rows_v)
        pltpu.sync_copy(rows_v, out_hbm.at[c, rs])


_ROWS = 400
_GRID = N_NODES // _ROWS


def _k0_body(x_ref, w_ref, degp_ref, dinv_ref, g_ref):
    deg = degp_ref[0] + degp_ref[1] + 1.0
    dinv = jnp.where(deg > 0, lax.rsqrt(deg), 0.0)
    dinv_ref[...] = dinv
    g_ref[...] = dinv * jnp.dot(x_ref[...], w_ref[...],
                                preferred_element_type=jnp.float32)


def _kmid_body(p_ref, gp_ref, dinv_ref, b_ref, w_ref, g_ref, *, relu):
    dinv = dinv_ref[...]
    h = dinv * (p_ref[0] + p_ref[1] + gp_ref[...]) + b_ref[...]
    if relu:
        h = jnp.maximum(h, 0.0)
    g_ref[...] = dinv * jnp.dot(h, w_ref[...],
                                preferred_element_type=jnp.float32)


def _kelem_body(p_ref, gp_ref, dinv_ref, b_ref, u_ref):
    dinv = dinv_ref[...]
    u_ref[...] = dinv * (dinv * (p_ref[0] + p_ref[1] + gp_ref[...])
                         + b_ref[...])


def _kfin_body(p_ref, u_ref, dinv_ref, w_ref, b_ref, o_ref):
    o_ref[...] = dinv_ref[...] * jnp.dot(
        p_ref[0] + p_ref[1] + u_ref[...], w_ref[...],
        preferred_element_type=jnp.float32) + b_ref[...]


def _row_spec(d):
    return pl.BlockSpec((_ROWS, d), lambda i: (i, 0))


def _part_spec(d):
    return pl.BlockSpec((NC, _ROWS, d), lambda i: (0, i, 0))


def _full_spec(r, c):
    return pl.BlockSpec((r, c), lambda i: (0, 0))


def _k0(x, w0, degp):
    return pl.pallas_call(
        _k0_body,
        grid=(_GRID,),
        in_specs=[_row_spec(D_HID), _full_spec(D_HID, D_HID), _part_spec(1)],
        out_specs=[_row_spec(1), _row_spec(D_HID)],
        out_shape=[jax.ShapeDtypeStruct((N_NODES, 1), jnp.float32),
                   jax.ShapeDtypeStruct((N_NODES, D_HID), jnp.float32)],
    )(x, w0, degp)


def _kmid(p, gp, dinv, b, w, relu):
    return pl.pallas_call(
        functools.partial(_kmid_body, relu=relu),
        grid=(_GRID,),
        in_specs=[_part_spec(D_HID), _row_spec(D_HID), _row_spec(1),
                  _full_spec(1, D_HID), _full_spec(D_HID, w.shape[1])],
        out_specs=_row_spec(w.shape[1]),
        out_shape=jax.ShapeDtypeStruct((N_NODES, w.shape[1]), jnp.float32),
    )(p, gp, dinv, b, w)


def _kelem(p, gp, dinv, b):
    return pl.pallas_call(
        _kelem_body,
        grid=(_GRID,),
        in_specs=[_part_spec(D_HID), _row_spec(D_HID), _row_spec(1),
                  _full_spec(1, D_HID)],
        out_specs=_row_spec(D_HID),
        out_shape=jax.ShapeDtypeStruct((N_NODES, D_HID), jnp.float32),
    )(p, gp, dinv, b)


def _kfin(p, u, dinv, w2, b2):
    ncls = w2.shape[1]
    return pl.pallas_call(
        _kfin_body,
        grid=(_GRID,),
        in_specs=[_part_spec(D_HID), _row_spec(D_HID), _row_spec(1),
                  _full_spec(D_HID, ncls), _full_spec(1, ncls)],
        out_specs=_row_spec(ncls),
        out_shape=jax.ShapeDtypeStruct((N_NODES, ncls), jnp.float32),
    )(p, u, dinv, w2, b2)


def kernel(x, edge_index, edge_attr, W0, b0, W1, b1, W2, b2):
    src = edge_index[0]
    dst = edge_index[1]
    ew = edge_attr

    n_chunks = N_EDGES // EK
    ed = jnp.stack(
        [src.reshape(n_chunks, EK), dst.reshape(n_chunks, EK),
         lax.bitcast_convert_type(ew, jnp.int32).reshape(n_chunks, EK)]
        + [jnp.zeros((n_chunks, EK), jnp.int32)] * 5,
        axis=1).reshape(n_chunks * 8, EK)

    zeros_n = jnp.zeros((N_NODES,), jnp.float32)
    zeros128 = jnp.zeros((EK, D_HID), jnp.float32)

    degp = _deg_kernel(dst, ew, zeros_n).reshape(NC, N_NODES, 1)

    dinv, g0 = _k0(x, W0, degp)
    p0 = _agg_kernel(g0, ed, zeros128)
    g1 = _kmid(p0, g0, dinv, b0.reshape(1, D_HID), W1, relu=True)
    p1 = _agg_kernel(g1, ed, zeros128)
    u2 = _kelem(p1, g1, dinv, b1.reshape(1, D_HID))
    p2 = _agg_kernel(u2, ed, zeros128)
    return _kfin(p2, u2, dinv, W2, b2.reshape(1, W2.shape[1]))


# speedup vs baseline: 1.9425x; 1.1064x over previous
import functools

import jax
import jax.numpy as jnp
from jax import lax
from jax.experimental import pallas as pl
from jax.experimental.pallas import tpu as pltpu
from jax.experimental.pallas import tpu_sc as plsc

N_NODES = 10000
N_EDGES = 320000
D_HID = 128

NC = 2
NS = 16
NW = NC * NS
EPW = N_EDGES // NW
EK = 80
N_PAD = 10240
ROWS_PER_SUB = N_PAD // NS

_sc_mesh = functools.partial(
    plsc.VectorSubcoreMesh, core_axis_name="c", subcore_axis_name="s")


@functools.partial(
    pl.kernel,
    mesh=_sc_mesh(),
    out_type=jax.ShapeDtypeStruct((NC * N_NODES,), jnp.float32),
    scratch_types=[
        pltpu.VMEM((EK,), jnp.int32),
        pltpu.VMEM((EK,), jnp.float32),
        pltpu.VMEM((N_NODES,), jnp.float32),
        pltpu.VMEM_SHARED((N_NODES,), jnp.float32),
    ],
)
def _deg_kernel(dst_hbm, ew_hbm, zeros_hbm, out_hbm, dst_v, w_v, deg_v,
                acc_sh):
    c = lax.axis_index("c")
    s = lax.axis_index("s")
    wid = s * NC + c

    @pl.when(s == 0)
    def _():
        pltpu.sync_copy(zeros_hbm, deg_v)
        pltpu.sync_copy(deg_v, acc_sh)

    plsc.subcore_barrier()

    def chunk(i, carry):
        base = wid * EPW + i * EK
        pltpu.sync_copy(dst_hbm.at[pl.ds(base, EK)], dst_v)
        pltpu.sync_copy(ew_hbm.at[pl.ds(base, EK)], w_v)
        pltpu.sync_copy(w_v, acc_sh.at[dst_v], add=True)
        return carry

    lax.fori_loop(0, EPW // EK, chunk, 0)
    plsc.subcore_barrier()

    @pl.when(s == 0)
    def _():
        pltpu.sync_copy(acc_sh, deg_v)
        pltpu.sync_copy(deg_v, out_hbm.at[pl.ds(c * N_NODES, N_NODES)])


@functools.partial(
    pl.kernel,
    mesh=_sc_mesh(),
    out_type=jax.ShapeDtypeStruct((NC, N_PAD, D_HID), jnp.float32),
    scratch_types=[
        pltpu.VMEM((2, 8, EK), jnp.int32),
        pltpu.VMEM((EK, D_HID), jnp.float32),
        pltpu.VMEM((EK, D_HID), jnp.float32),
        pltpu.VMEM_SHARED((N_PAD, D_HID), jnp.float32),
        pltpu.SemaphoreType.DMA((2,)),
        pltpu.SemaphoreType.DMA,
        pltpu.SemaphoreType.DMA,
    ],
)
def _agg_kernel(g_hbm, ed_hbm, zeros_hbm, out_hbm,
                ed2_v, rows_v, rows_s, acc_sh, isem, sem, ssem):
    c = lax.axis_index("c")
    s = lax.axis_index("s")
    wid = s * NC + c
    n_slab = ROWS_PER_SUB // EK
    pltpu.sync_copy(zeros_hbm, rows_v)
    for t in range(n_slab):
        pltpu.sync_copy(
            rows_v, acc_sh.at[pl.ds(s * ROWS_PER_SUB + t * EK, EK)])
    plsc.subcore_barrier()

    def idx_dma(ci, b):
        crow = (wid * (EPW // EK) + ci) * 8
        return pltpu.make_async_copy(
            ed_hbm.at[pl.ds(crow, 8)], ed2_v.at[b], isem.at[b])

    idx_dma(0, 0).start()

    def chunk(i, carry):
        b = lax.rem(i, 2)

        @pl.when(i + 1 < EPW // EK)
        def _():
            idx_dma(i + 1, 1 - b).start()

        idx_dma(i, b).wait()
        ed_v = ed2_v.at[b]
        pltpu.async_copy(g_hbm.at[ed_v.at[0]], rows_v, sem).wait()

        @pl.when(i >= 1)
        def _():
            pltpu.make_async_copy(rows_s, acc_sh.at[ed_v.at[1]],
                                  ssem).wait()

        def scale(g, cc):
            wv = lax.bitcast_convert_type(
                ed2_v[b, 2, pl.ds(g * 16, 16)], jnp.float32)
            for k in range(16):
                e = g * 16 + k
                w = wv[k]
                for j in range(D_HID // 16):
                    sl = pl.ds(j * 16, 16)
                    rows_s[e, sl] = rows_v[e, sl] * w
            return cc

        lax.fori_loop(0, EK // 16, scale, 0)
        pltpu.async_copy(rows_s, acc_sh.at[ed_v.at[1]], ssem, add=True)
        return carry

    lax.fori_loop(0, EPW // EK, chunk, 0)
    pltpu.make_async_copy(
        rows_s, acc_sh.at[ed2_v.at[0].at[1]], ssem).wait()
    plsc.subcore_barrier()
    for t in range(n_slab):
        rs = pl.ds(s * ROWS_PER_SUB + t * EK, EK)
        pltpu.sync_copy(acc_sh.at[rs], rows_v)
        pltpu.sync_copy(rows_v, out_hbm.at[c, rs])


_ROWS = 400
_GRID = N_NODES // _ROWS


def _k0_body(x_ref, w_ref, degp_ref, dinv_ref, g_ref):
    deg = degp_ref[0] + degp_ref[1] + 1.0
    dinv = jnp.where(deg > 0, lax.rsqrt(deg), 0.0)
    dinv_ref[...] = dinv
    g_ref[...] = dinv * jnp.dot(x_ref[...], w_ref[...],
                                preferred_element_type=jnp.float32)


def _kmid_body(p_ref, gp_ref, dinv_ref, b_ref, w_ref, g_ref, *, relu):
    dinv = dinv_ref[...]
    h = dinv * (p_ref[0] + p_ref[1] + gp_ref[...]) + b_ref[...]
    if relu:
        h = jnp.maximum(h, 0.0)
    g_ref[...] = dinv * jnp.dot(h, w_ref[...],
                                preferred_element_type=jnp.float32)


def _kelem_body(p_ref, gp_ref, dinv_ref, b_ref, u_ref):
    dinv = dinv_ref[...]
    u_ref[...] = dinv * (dinv * (p_ref[0] + p_ref[1] + gp_ref[...])
                         + b_ref[...])


def _kfin_body(p_ref, u_ref, dinv_ref, w_ref, b_ref, o_ref):
    o_ref[...] = dinv_ref[...] * jnp.dot(
        p_ref[0] + p_ref[1] + u_ref[...], w_ref[...],
        preferred_element_type=jnp.float32) + b_ref[...]


def _row_spec(d):
    return pl.BlockSpec((_ROWS, d), lambda i: (i, 0))


def _part_spec(d):
    return pl.BlockSpec((NC, _ROWS, d), lambda i: (0, i, 0))


def _full_spec(r, c):
    return pl.BlockSpec((r, c), lambda i: (0, 0))


def _k0(x, w0, degp):
    return pl.pallas_call(
        _k0_body,
        grid=(_GRID,),
        in_specs=[_row_spec(D_HID), _full_spec(D_HID, D_HID), _part_spec(1)],
        out_specs=[_row_spec(1), _row_spec(D_HID)],
        out_shape=[jax.ShapeDtypeStruct((N_NODES, 1), jnp.float32),
                   jax.ShapeDtypeStruct((N_NODES, D_HID), jnp.float32)],
    )(x, w0, degp)


def _kmid(p, gp, dinv, b, w, relu):
    return pl.pallas_call(
        functools.partial(_kmid_body, relu=relu),
        grid=(_GRID,),
        in_specs=[_part_spec(D_HID), _row_spec(D_HID), _row_spec(1),
                  _full_spec(1, D_HID), _full_spec(D_HID, w.shape[1])],
        out_specs=_row_spec(w.shape[1]),
        out_shape=jax.ShapeDtypeStruct((N_NODES, w.shape[1]), jnp.float32),
    )(p, gp, dinv, b, w)


def _kelem(p, gp, dinv, b):
    return pl.pallas_call(
        _kelem_body,
        grid=(_GRID,),
        in_specs=[_part_spec(D_HID), _row_spec(D_HID), _row_spec(1),
                  _full_spec(1, D_HID)],
        out_specs=_row_spec(D_HID),
        out_shape=jax.ShapeDtypeStruct((N_NODES, D_HID), jnp.float32),
    )(p, gp, dinv, b)


def _kfin(p, u, dinv, w2, b2):
    ncls = w2.shape[1]
    return pl.pallas_call(
        _kfin_body,
        grid=(_GRID,),
        in_specs=[_part_spec(D_HID), _row_spec(D_HID), _row_spec(1),
                  _full_spec(D_HID, ncls), _full_spec(1, ncls)],
        out_specs=_row_spec(ncls),
        out_shape=jax.ShapeDtypeStruct((N_NODES, ncls), jnp.float32),
    )(p, u, dinv, w2, b2)


def kernel(x, edge_index, edge_attr, W0, b0, W1, b1, W2, b2):
    src = edge_index[0]
    dst = edge_index[1]
    ew = edge_attr

    n_chunks = N_EDGES // EK
    ed = jnp.stack(
        [src.reshape(n_chunks, EK), dst.reshape(n_chunks, EK),
         lax.bitcast_convert_type(ew, jnp.int32).reshape(n_chunks, EK)]
        + [jnp.zeros((n_chunks, EK), jnp.int32)] * 5,
        axis=1).reshape(n_chunks * 8, EK)

    zeros_n = jnp.zeros((N_NODES,), jnp.float32)
    zeros128 = jnp.zeros((EK, D_HID), jnp.float32)

    degp = _deg_kernel(dst, ew, zeros_n).reshape(NC, N_NODES, 1)

    dinv, g0 = _k0(x, W0, degp)
    p0 = _agg_kernel(g0, ed, zeros128)
    g1 = _kmid(p0, g0, dinv, b0.reshape(1, D_HID), W1, relu=True)
    p1 = _agg_kernel(g1, ed, zeros128)
    u2 = _kelem(p1, g1, dinv, b1.reshape(1, D_HID))
    p2 = _agg_kernel(u2, ed, zeros128)
    return _kfin(p2, u2, dinv, W2, b2.reshape(1, W2.shape[1]))


# speedup vs baseline: 2.0046x; 1.0320x over previous
import functools

import jax
import jax.numpy as jnp
from jax import lax
from jax.experimental import pallas as pl
from jax.experimental.pallas import tpu as pltpu
from jax.experimental.pallas import tpu_sc as plsc

N_NODES = 10000
N_EDGES = 320000
D_HID = 128

NC = 2
NS = 16
NW = NC * NS
EPW = N_EDGES // NW
EK = 80
N_PAD = 10240
ROWS_PER_SUB = N_PAD // NS

_sc_mesh = functools.partial(
    plsc.VectorSubcoreMesh, core_axis_name="c", subcore_axis_name="s")


@functools.partial(
    pl.kernel,
    mesh=_sc_mesh(),
    out_type=jax.ShapeDtypeStruct((NC * N_NODES,), jnp.float32),
    scratch_types=[
        pltpu.VMEM((EK,), jnp.int32),
        pltpu.VMEM((EK,), jnp.float32),
        pltpu.VMEM((N_NODES,), jnp.float32),
        pltpu.VMEM_SHARED((N_NODES,), jnp.float32),
    ],
)
def _deg_kernel(dst_hbm, ew_hbm, zeros_hbm, out_hbm, dst_v, w_v, deg_v,
                acc_sh):
    c = lax.axis_index("c")
    s = lax.axis_index("s")
    wid = s * NC + c

    @pl.when(s == 0)
    def _():
        pltpu.sync_copy(zeros_hbm, deg_v)
        pltpu.sync_copy(deg_v, acc_sh)

    plsc.subcore_barrier()

    def chunk(i, carry):
        base = wid * EPW + i * EK
        pltpu.sync_copy(dst_hbm.at[pl.ds(base, EK)], dst_v)
        pltpu.sync_copy(ew_hbm.at[pl.ds(base, EK)], w_v)
        pltpu.sync_copy(w_v, acc_sh.at[dst_v], add=True)
        return carry

    lax.fori_loop(0, EPW // EK, chunk, 0)
    plsc.subcore_barrier()

    @pl.when(s == 0)
    def _():
        pltpu.sync_copy(acc_sh, deg_v)
        pltpu.sync_copy(deg_v, out_hbm.at[pl.ds(c * N_NODES, N_NODES)])


@functools.partial(
    pl.kernel,
    mesh=_sc_mesh(),
    out_type=jax.ShapeDtypeStruct((NC, N_PAD, D_HID), jnp.float32),
    scratch_types=[
        pltpu.VMEM((2, 8, EK), jnp.int32),
        pltpu.VMEM((EK, D_HID), jnp.float32),
        pltpu.VMEM((EK, D_HID), jnp.float32),
        pltpu.VMEM_SHARED((N_PAD, D_HID), jnp.float32),
        pltpu.SemaphoreType.DMA((2,)),
        pltpu.SemaphoreType.DMA,
        pltpu.SemaphoreType.DMA,
        pltpu.SemaphoreType.DMA,
    ],
)
def _agg_kernel(g_hbm, ed_hbm, zeros_hbm, out_hbm,
                ed2_v, rows_v, rows_s, acc_sh, isem, sem, sem2, ssem):
    c = lax.axis_index("c")
    s = lax.axis_index("s")
    wid = s * NC + c
    n_slab = ROWS_PER_SUB // EK
    pltpu.sync_copy(zeros_hbm, rows_v)
    for t in range(n_slab):
        pltpu.sync_copy(
            rows_v, acc_sh.at[pl.ds(s * ROWS_PER_SUB + t * EK, EK)])
    plsc.subcore_barrier()

    def idx_dma(ci, b):
        crow = (wid * (EPW // EK) + ci) * 8
        return pltpu.make_async_copy(
            ed_hbm.at[pl.ds(crow, 8)], ed2_v.at[b], isem.at[b])

    idx_dma(0, 0).start()

    def chunk(i, carry):
        b = lax.rem(i, 2)

        @pl.when(i + 1 < EPW // EK)
        def _():
            idx_dma(i + 1, 1 - b).start()

        idx_dma(i, b).wait()
        ed_v = ed2_v.at[b]
        h1 = pltpu.async_copy(
            g_hbm.at[ed2_v.at[b, 0, pl.ds(0, EK // 2)]],
            rows_v.at[pl.ds(0, EK // 2)], sem)
        h2 = pltpu.async_copy(
            g_hbm.at[ed2_v.at[b, 0, pl.ds(EK // 2, EK // 2)]],
            rows_v.at[pl.ds(EK // 2, EK // 2)], sem2)
        h1.wait()
        h2.wait()

        @pl.when(i >= 1)
        def _():
            pltpu.make_async_copy(rows_s, acc_sh.at[ed_v.at[1]],
                                  ssem).wait()

        def scale(g, cc):
            wv = lax.bitcast_convert_type(
                ed2_v[b, 2, pl.ds(g * 16, 16)], jnp.float32)
            for k in range(16):
                e = g * 16 + k
                w = wv[k]
                for j in range(D_HID // 16):
                    sl = pl.ds(j * 16, 16)
                    rows_s[e, sl] = rows_v[e, sl] * w
            return cc

        lax.fori_loop(0, EK // 16, scale, 0)
        pltpu.async_copy(rows_s, acc_sh.at[ed_v.at[1]], ssem, add=True)
        return carry

    lax.fori_loop(0, EPW // EK, chunk, 0)
    pltpu.make_async_copy(
        rows_s, acc_sh.at[ed2_v.at[0].at[1]], ssem).wait()
    plsc.subcore_barrier()
    for t in range(n_slab):
        rs = pl.ds(s * ROWS_PER_SUB + t * EK, EK)
        pltpu.sync_copy(acc_sh.at[rs], rows_v)
        pltpu.sync_copy(rows_v, out_hbm.at[c, rs])


_ROWS = 400
_GRID = N_NODES // _ROWS


def _k0_body(x_ref, w_ref, degp_ref, dinv_ref, g_ref):
    deg = degp_ref[0] + degp_ref[1] + 1.0
    dinv = jnp.where(deg > 0, lax.rsqrt(deg), 0.0)
    dinv_ref[...] = dinv
    g_ref[...] = dinv * jnp.dot(x_ref[...], w_ref[...],
                                preferred_element_type=jnp.float32)


def _kmid_body(p_ref, gp_ref, dinv_ref, b_ref, w_ref, g_ref, *, relu):
    dinv = dinv_ref[...]
    h = dinv * (p_ref[0] + p_ref[1] + gp_ref[...]) + b_ref[...]
    if relu:
        h = jnp.maximum(h, 0.0)
    g_ref[...] = dinv * jnp.dot(h, w_ref[...],
                                preferred_element_type=jnp.float32)


def _kelem_body(p_ref, gp_ref, dinv_ref, b_ref, u_ref):
    dinv = dinv_ref[...]
    u_ref[...] = dinv * (dinv * (p_ref[0] + p_ref[1] + gp_ref[...])
                         + b_ref[...])


def _kfin_body(p_ref, u_ref, dinv_ref, w_ref, b_ref, o_ref):
    o_ref[...] = dinv_ref[...] * jnp.dot(
        p_ref[0] + p_ref[1] + u_ref[...], w_ref[...],
        preferred_element_type=jnp.float32) + b_ref[...]


def _row_spec(d):
    return pl.BlockSpec((_ROWS, d), lambda i: (i, 0))


def _part_spec(d):
    return pl.BlockSpec((NC, _ROWS, d), lambda i: (0, i, 0))


def _full_spec(r, c):
    return pl.BlockSpec((r, c), lambda i: (0, 0))


def _k0(x, w0, degp):
    return pl.pallas_call(
        _k0_body,
        grid=(_GRID,),
        in_specs=[_row_spec(D_HID), _full_spec(D_HID, D_HID), _part_spec(1)],
        out_specs=[_row_spec(1), _row_spec(D_HID)],
        out_shape=[jax.ShapeDtypeStruct((N_NODES, 1), jnp.float32),
                   jax.ShapeDtypeStruct((N_NODES, D_HID), jnp.float32)],
    )(x, w0, degp)


def _kmid(p, gp, dinv, b, w, relu):
    return pl.pallas_call(
        functools.partial(_kmid_body, relu=relu),
        grid=(_GRID,),
        in_specs=[_part_spec(D_HID), _row_spec(D_HID), _row_spec(1),
                  _full_spec(1, D_HID), _full_spec(D_HID, w.shape[1])],
        out_specs=_row_spec(w.shape[1]),
        out_shape=jax.ShapeDtypeStruct((N_NODES, w.shape[1]), jnp.float32),
    )(p, gp, dinv, b, w)


def _kelem(p, gp, dinv, b):
    return pl.pallas_call(
        _kelem_body,
        grid=(_GRID,),
        in_specs=[_part_spec(D_HID), _row_spec(D_HID), _row_spec(1),
                  _full_spec(1, D_HID)],
        out_specs=_row_spec(D_HID),
        out_shape=jax.ShapeDtypeStruct((N_NODES, D_HID), jnp.float32),
    )(p, gp, dinv, b)


def _kfin(p, u, dinv, w2, b2):
    ncls = w2.shape[1]
    return pl.pallas_call(
        _kfin_body,
        grid=(_GRID,),
        in_specs=[_part_spec(D_HID), _row_spec(D_HID), _row_spec(1),
                  _full_spec(D_HID, ncls), _full_spec(1, ncls)],
        out_specs=_row_spec(ncls),
        out_shape=jax.ShapeDtypeStruct((N_NODES, ncls), jnp.float32),
    )(p, u, dinv, w2, b2)


def kernel(x, edge_index, edge_attr, W0, b0, W1, b1, W2, b2):
    src = edge_index[0]
    dst = edge_index[1]
    ew = edge_attr

    n_chunks = N_EDGES // EK
    ed = jnp.stack(
        [src.reshape(n_chunks, EK), dst.reshape(n_chunks, EK),
         lax.bitcast_convert_type(ew, jnp.int32).reshape(n_chunks, EK)]
        + [jnp.zeros((n_chunks, EK), jnp.int32)] * 5,
        axis=1).reshape(n_chunks * 8, EK)

    zeros_n = jnp.zeros((N_NODES,), jnp.float32)
    zeros128 = jnp.zeros((EK, D_HID), jnp.float32)

    degp = _deg_kernel(dst, ew, zeros_n).reshape(NC, N_NODES, 1)

    dinv, g0 = _k0(x, W0, degp)
    p0 = _agg_kernel(g0, ed, zeros128)
    g1 = _kmid(p0, g0, dinv, b0.reshape(1, D_HID), W1, relu=True)
    p1 = _agg_kernel(g1, ed, zeros128)
    u2 = _kelem(p1, g1, dinv, b1.reshape(1, D_HID))
    p2 = _agg_kernel(u2, ed, zeros128)
    return _kfin(p2, u2, dinv, W2, b2.reshape(1, W2.shape[1]))
